# baseline (speedup 1.0000x reference)
import jax
import jax.numpy as jnp
from jax.experimental import pallas as pl
from jax.experimental.pallas import tpu as pltpu

N_ROWS = 10000
K = 128
N_OUT = 75
BLOCK_M = 2000


def _linear_kernel(z_ref, w_ref, b_ref, out_ref):
    acc = jax.lax.dot_general(
        z_ref[...], w_ref[...],
        dimension_numbers=(((1,), (1,)), ((), ())),
        preferred_element_type=jnp.float32,
    )
    out_ref[...] = acc + b_ref[...]


def kernel(z, edge_index, weight, sim, W, b):
    del edge_index, weight, sim
    b2 = b.reshape(1, N_OUT)
    z = pltpu.with_memory_space_constraint(z, pltpu.MemorySpace.HBM)
    out = pl.pallas_call(
        _linear_kernel,
        grid=(N_ROWS // BLOCK_M,),
        in_specs=[
            pl.BlockSpec((BLOCK_M, K), lambda i: (i, 0)),
            pl.BlockSpec((N_OUT, K), lambda i: (0, 0)),
            pl.BlockSpec((1, N_OUT), lambda i: (0, 0)),
        ],
        out_specs=pl.BlockSpec((BLOCK_M, N_OUT), lambda i: (i, 0)),
        out_shape=jax.ShapeDtypeStruct((N_ROWS, N_OUT), jnp.float32),
    )(z, W, b2)
    return out.reshape(-1)


# speedup vs baseline: 1.1777x; 1.1777x over previous
import jax
import jax.numpy as jnp
from jax.experimental import pallas as pl
from jax.experimental.pallas import tpu as pltpu

N_ROWS = 10000
K = 128
N_OUT = 75
BLOCK_M = 5000


def _linear_kernel(z_ref, w_ref, b_ref, out_ref):
    acc = jax.lax.dot_general(
        z_ref[...], w_ref[...],
        dimension_numbers=(((1,), (1,)), ((), ())),
        preferred_element_type=jnp.float32,
    )
    out_ref[...] = acc + b_ref[...]


def kernel(z, edge_index, weight, sim, W, b):
    del edge_index, weight, sim
    b2 = b.reshape(1, N_OUT)
    z = pltpu.with_memory_space_constraint(z, pltpu.MemorySpace.HBM)
    out = pl.pallas_call(
        _linear_kernel,
        grid=(N_ROWS // BLOCK_M,),
        in_specs=[
            pl.BlockSpec((BLOCK_M, K), lambda i: (i, 0)),
            pl.BlockSpec((N_OUT, K), lambda i: (0, 0)),
            pl.BlockSpec((1, N_OUT), lambda i: (0, 0)),
        ],
        out_specs=pl.BlockSpec((BLOCK_M, N_OUT), lambda i: (i, 0)),
        out_shape=jax.ShapeDtypeStruct((N_ROWS, N_OUT), jnp.float32),
    )(z, W, b2)
    return out.reshape(-1)
